# baseline (speedup 1.0000x reference)
import functools

import jax
import jax.numpy as jnp
from jax import lax
from jax.experimental import pallas as pl
from jax.experimental.pallas import tpu as pltpu
from jax.experimental.pallas import tpu_sc as plsc

NC = 2
NS = 16
L = 16

B = 128
NSLOT = 32
F = 2 * NSLOT


@functools.lru_cache(maxsize=None)
def _build(E, N, D):
    assert E % B == 0 and D % L == 0 and N % NC == 0
    NSEG = N // NC
    DUMP = NSEG
    ATT0 = NSEG + 8
    APACK = -(-NSEG // (8 * 8)) * 8
    ACC_ROWS = ATT0 + APACK
    RPT = -(-NSEG // (NS * 8)) * 8
    FCH = 64
    NCHUNK = -(-RPT // FCH)
    NZCH = -(-ACC_ROWS // (B * NS))
    assert NSEG % 8 == 0 and ATT0 % 8 == 0 and (NSEG - FCH) % 8 == 0

    mesh = plsc.VectorSubcoreMesh(core_axis_name="c", subcore_axis_name="s",
                                  num_cores=NC, num_subcores=NS)

    @functools.partial(
        pl.kernel,
        out_type=jax.ShapeDtypeStruct((N, D), jnp.float32),
        mesh=mesh,
        scratch_types=[
            pltpu.VMEM_SHARED((ACC_ROWS, D), jnp.float32),
            [pltpu.VMEM((B, D), jnp.float32)] * 2,
            pltpu.VMEM((F, D), jnp.float32),
            pltpu.VMEM((F,), jnp.int32),
            [pltpu.VMEM((B,), jnp.int32)] * 2,
            [pltpu.VMEM((B,), jnp.float32)] * 2,
            pltpu.VMEM((NSLOT * L,), jnp.float32),
            pltpu.VMEM((NSLOT * L,), jnp.int32),
            pltpu.VMEM((L,), jnp.int32),
            [pltpu.SemaphoreType.DMA] * 2,
        ],
    )
    def sc_kernel(emap_hbm, att_hbm, y_hbm, cut_hbm, out_hbm,
                  acc, ybuf, fbuf, fidx, ebuf, abuf, cabuf, cidx,
                  cutv, sem_in):
        c = lax.axis_index("c")
        t = lax.axis_index("s")

        pltpu.sync_copy(cut_hbm, cutv)
        cut = cutv[...][0]
        lo = jnp.where(c == 0, 0, cut)
        hi = jnp.where(c == 0, cut, E)
        base = c * NSEG

        zv = jnp.zeros((L,), jnp.float32)
        dumpv = jnp.full((L,), DUMP, jnp.int32)
        iov = lax.iota(jnp.int32, L)

        def zfb(i, _):
            for j in range(D // L):
                fbuf[i, pl.ds(j * L, L)] = zv
            return 0
        lax.fori_loop(0, F, zfb, 0)
        for q in range(F // L):
            fidx[pl.ds(q * L, L)] = dumpv

        def zca(i, _):
            cabuf[pl.ds(i * L, L)] = zv
            cidx[pl.ds(i * L, L)] = jnp.zeros((L,), jnp.int32)
            return 0
        lax.fori_loop(0, NSLOT, zca, 0)

        def zyb(i, _):
            for j in range(D // L):
                ybuf[0][i, pl.ds(j * L, L)] = zv
            return 0
        lax.fori_loop(0, B, zyb, 0)
        for kk in range(NZCH):
            q = t + kk * NS
            zr = jnp.minimum(q * B, ACC_ROWS - B)
            pltpu.sync_copy(ybuf[0], acc.at[pl.ds(zr, B)])
        plsc.subcore_barrier()

        g0 = lax.div(lo, B)
        g1 = lax.div(hi + (B - 1), B)
        per = lax.div(g1 - g0 + (NS - 1), NS)
        bs = g0 + t * per
        be = jnp.maximum(jnp.minimum(bs + per, g1), bs)
        nb = be - bs

        def issue_in(g, w):
            eb = g * B
            pltpu.async_copy(y_hbm.at[pl.ds(eb, B)], ybuf[w], sem_in[w])
            pltpu.async_copy(emap_hbm.at[pl.ds(eb, B)], ebuf[w], sem_in[w])
            pltpu.async_copy(att_hbm.at[pl.ds(eb, B)], abuf[w], sem_in[w])

        def wait_in(w):
            pltpu.make_async_copy(y_hbm.at[pl.ds(0, B)], ybuf[w],
                                  sem_in[w]).wait()
            pltpu.make_async_copy(emap_hbm.at[pl.ds(0, B)], ebuf[w],
                                  sem_in[w]).wait()
            pltpu.make_async_copy(att_hbm.at[pl.ds(0, B)], abuf[w],
                                  sem_in[w]).wait()

        def scatter_reset(nf):
            zvl = jnp.zeros((L,), jnp.float32)
            dumpl = jnp.full((L,), DUMP, jnp.int32)
            for k in range(NSLOT):
                cav = cabuf[pl.ds(k * L, L)]
                col = cidx[pl.ds(k * L, L)][0]
                fbuf[2 * k + 1, pl.ds(col, L)] = cav
            pltpu.sync_copy(fbuf, acc.at[fidx], add=True)
            for q in range(F // L):
                fidx[pl.ds(q * L, L)] = dumpl

            def zatt(i, _):
                for j in range(D // L):
                    fbuf[2 * i + 1, pl.ds(j * L, L)] = zvl
                return 0
            lax.fori_loop(0, NSLOT, zatt, 0)
            return jnp.int32(-1)

        def maybe_scatter(nf):
            return lax.cond(nf > NSLOT - 18, scatter_reset, lambda x: x, nf)

        def process_block(g, live, u, nf):
            hil = jnp.where(live, hi, -1)
            def emit_slot(slot, seg, cav):
                cabuf[pl.ds(slot * L, L)] = cav
                cidx[pl.ds(slot * L, L)] = jnp.broadcast_to(
                    (seg % 8) * L, (L,))
                pos = 2 * slot
                grp16 = lax.div(pos, L) * L
                lane = pos - grp16
                attidx = ATT0 + lax.div(seg, 8)
                iv = fidx[pl.ds(grp16, L)]
                niv = jnp.where(iov == lane, seg,
                                jnp.where(iov == lane + 1, attidx, iv))
                fidx[pl.ds(grp16, L)] = niv

            def grp(i, nf):
                off = i * L
                ev = ebuf[u][pl.ds(off, L)]
                av = abuf[u][pl.ds(off, L)]
                gidx = g * B + off + iov
                ok = (gidx >= lo) & (gidx < hil)
                s = jnp.where(ok, jnp.exp(av), jnp.float32(0.0))
                loc = jnp.clip(ev - base, 0, NSEG - 1)
                prev = None
                cj = None
                ca = None
                for r in range(L):
                    lr = loc[r]
                    sb = jnp.broadcast_to(s[r], (L,))
                    row = off + r
                    if r == 0:
                        nf = nf + 1
                        ncj = [ybuf[u][row, pl.ds(j * L, L)] * sb
                               for j in range(D // L)]
                        ca = sb
                    else:
                        b = lr != prev
                        nfp = nf
                        nf = nf + b.astype(jnp.int32)
                        km = jnp.where(b, jnp.float32(0.0), jnp.float32(1.0))

                        @pl.when(b)
                        def _(prev=prev, ca=ca, nfp=nfp):
                            emit_slot(nfp, prev, ca)
                        ncj = [cj[j] * km + ybuf[u][row, pl.ds(j * L, L)] * sb
                               for j in range(D // L)]
                        ca = ca * km + sb
                    for j in range(D // L):
                        fbuf[2 * nf, pl.ds(j * L, L)] = ncj[j]
                    cj = ncj
                    prev = lr
                emit_slot(nf, prev, ca)
                return maybe_scatter(nf)
            return lax.fori_loop(0, B // L, grp, nf)

        npairs = lax.div(nb + 1, 2)
        emax = jnp.maximum(be - 1, 0)

        def eff(g):
            return jnp.clip(g, 0, emax)

        @pl.when(nb > 0)
        def _():
            issue_in(bs, 0)

        def pair(p, nf):
            for u in (0, 1):
                g = bs + 2 * p + u
                if u == 0:
                    issue_in(eff(g + 1), 1)
                else:
                    @pl.when(p < npairs - 1)
                    def _():
                        issue_in(eff(g + 1), 0)
                wait_in(u)
                nf = process_block(eff(g), g < be, u, nf)
            return nf

        nf = lax.fori_loop(0, npairs, pair, jnp.int32(-1))

        _ = lax.cond(nf >= 0, scatter_reset, lambda x: x, nf)
        plsc.subcore_barrier()

        r0 = t * RPT
        for k in range(NCHUNK):
            rr = jnp.minimum(r0 + k * FCH, NSEG - FCH)
            pltpu.sync_copy(acc.at[pl.ds(rr, FCH)], ybuf[0].at[pl.ds(0, FCH)])
            ar = ATT0 + lax.div(rr, 8)
            pltpu.sync_copy(acc.at[pl.ds(ar, FCH // 8)],
                            ybuf[1].at[pl.ds(0, FCH // 8)])

            def fingrp(gi, _):
                for r in range(L):
                    av = ybuf[1][gi * 2 + r // 8, pl.ds((r % 8) * L, L)]
                    rv = jnp.float32(1.0) / av
                    row = gi * L + r
                    for j in range(D // L):
                        ybuf[0][row, pl.ds(j * L, L)] = (
                            ybuf[0][row, pl.ds(j * L, L)] * rv)
                return 0
            lax.fori_loop(0, FCH // L, fingrp, 0)
            pltpu.sync_copy(ybuf[0].at[pl.ds(0, FCH)],
                            out_hbm.at[pl.ds(base + rr, FCH)])

    return sc_kernel


def kernel(X_in, e_map, v_count, Y, Y_att):
    E, D = Y.shape
    N = v_count.shape[0]
    att = Y_att.reshape((E,))
    cut = jnp.searchsorted(e_map, jnp.int32(N // NC)).astype(jnp.int32)
    cut_arr = jnp.full((L,), cut, dtype=jnp.int32)
    return _build(E, N, D)(e_map, att, Y, cut_arr)


# speedup vs baseline: 1.4233x; 1.4233x over previous
import functools

import jax
import jax.numpy as jnp
from jax import lax
from jax.experimental import pallas as pl
from jax.experimental.pallas import tpu as pltpu
from jax.experimental.pallas import tpu_sc as plsc

NC = 2
NS = 16
L = 16

B = 64


@functools.lru_cache(maxsize=None)
def _build(E, N, D):
    assert E % B == 0 and D % L == 0 and N % NC == 0
    NSEG = N // NC
    RPT = -(-NSEG // (NS * 8)) * 8
    NCHUNK = -(-RPT // B)
    assert NSEG % 8 == 0 and (NSEG - B) % 8 == 0

    mesh = plsc.VectorSubcoreMesh(core_axis_name="c", subcore_axis_name="s",
                                  num_cores=NC, num_subcores=NS)

    @functools.partial(
        pl.kernel,
        out_type=jax.ShapeDtypeStruct((N, D), jnp.float32),
        mesh=mesh,
        scratch_types=[
            pltpu.VMEM_SHARED((NSEG, D), jnp.float32),
            pltpu.VMEM_SHARED((NSEG, D), jnp.float32),
            [pltpu.VMEM((B, D), jnp.float32)] * 2,
            [pltpu.VMEM((B, D), jnp.float32)] * 2,
            [pltpu.VMEM((B, D), jnp.float32)] * 2,
            [pltpu.VMEM((B,), jnp.int32)] * 2,
            [pltpu.VMEM((B,), jnp.float32)] * 2,
            [pltpu.VMEM((B,), jnp.int32)] * 2,
            pltpu.VMEM((L,), jnp.int32),
            [pltpu.SemaphoreType.DMA] * 2,
            [pltpu.SemaphoreType.DMA] * 2,
        ],
    )
    def sc_kernel(emap_hbm, att_hbm, y_hbm, cut_hbm, out_hbm,
                  acc, aacc, ybuf, ymst, astage, ebuf, abuf, idxbuf,
                  cutv, sem_in, sem_out):
        c = lax.axis_index("c")
        t = lax.axis_index("s")

        pltpu.sync_copy(cut_hbm, cutv)
        cut = cutv[...][0]
        lo = jnp.where(c == 0, 0, cut)
        hi = jnp.where(c == 0, cut, E)
        base = c * NSEG

        g0 = lax.div(lo, B)
        g1 = lax.div(hi + (B - 1), B)
        per = lax.div(g1 - g0 + (NS - 1), NS)
        bs = g0 + t * per
        be = jnp.maximum(jnp.minimum(bs + per, g1), bs)
        nb = be - bs

        def issue_in(g, w):
            eb = g * B
            pltpu.async_copy(y_hbm.at[pl.ds(eb, B)], ybuf[w], sem_in[w])
            pltpu.async_copy(emap_hbm.at[pl.ds(eb, B)], ebuf[w], sem_in[w])
            pltpu.async_copy(att_hbm.at[pl.ds(eb, B)], abuf[w], sem_in[w])

        @pl.when(nb > 0)
        def _():
            issue_in(bs, 0)

        zv = jnp.zeros((L,), jnp.float32)

        def zrow(i, _):
            for j in range(D // L):
                ymst[0][i, pl.ds(j * L, L)] = zv
                astage[0][i, pl.ds(j * L, L)] = zv
                astage[1][i, pl.ds(j * L, L)] = zv
            return 0
        lax.fori_loop(0, B, zrow, 0)

        r0 = t * RPT
        for k in range(NCHUNK):
            rr = jnp.minimum(r0 + k * B, NSEG - B)
            pltpu.async_copy(ymst[0], acc.at[pl.ds(rr, B)], sem_out[0])
            pltpu.async_copy(ymst[0], aacc.at[pl.ds(rr, B)], sem_out[0])
        for k in range(NCHUNK):
            rr = jnp.minimum(r0 + k * B, NSEG - B)
            pltpu.make_async_copy(ymst[0], acc.at[pl.ds(rr, B)],
                                  sem_out[0]).wait()
            pltpu.make_async_copy(ymst[0], aacc.at[pl.ds(rr, B)],
                                  sem_out[0]).wait()
        plsc.subcore_barrier()

        def wait_in(w):
            pltpu.make_async_copy(y_hbm.at[pl.ds(0, B)], ybuf[w],
                                  sem_in[w]).wait()
            pltpu.make_async_copy(emap_hbm.at[pl.ds(0, B)], ebuf[w],
                                  sem_in[w]).wait()
            pltpu.make_async_copy(att_hbm.at[pl.ds(0, B)], abuf[w],
                                  sem_in[w]).wait()

        def issue_out(w):
            pltpu.async_copy(ymst[w], acc.at[idxbuf[w]], sem_out[w],
                             add=True)
            pltpu.async_copy(astage[w], aacc.at[idxbuf[w]], sem_out[w],
                             add=True)

        def wait_out(w):
            pltpu.make_async_copy(ymst[w], acc.at[idxbuf[w]],
                                  sem_out[w]).wait()
            pltpu.make_async_copy(astage[w], aacc.at[idxbuf[w]],
                                  sem_out[w]).wait()

        def compute(g, w):
            eb = g * B

            def grp(i, _):
                off = i * L
                ev = ebuf[w][pl.ds(off, L)]
                av = abuf[w][pl.ds(off, L)]
                gidx = eb + off + lax.iota(jnp.int32, L)
                ok = (gidx >= lo) & (gidx < hi)
                s = jnp.where(ok, jnp.exp(av), jnp.float32(0.0))
                idxbuf[w][pl.ds(off, L)] = jnp.clip(ev - base, 0, NSEG - 1)
                for r in range(L):
                    sb = jnp.broadcast_to(s[r], (L,))
                    row = off + r
                    astage[w][row, pl.ds(0, L)] = sb
                    for j in range(D // L):
                        ymst[w][row, pl.ds(j * L, L)] = (
                            ybuf[w][row, pl.ds(j * L, L)] * sb)
                return 0
            lax.fori_loop(0, B // L, grp, 0)

        def pair(p, _):
            for u in (0, 1):
                i = 2 * p + u
                g = bs + i

                @pl.when(g < be)
                def _():
                    @pl.when(g + 1 < be)
                    def _():
                        issue_in(g + 1, 1 - u)
                    wait_in(u)

                    @pl.when(i >= 2)
                    def _():
                        wait_out(u)
                    compute(g, u)
                    issue_out(u)
            return 0
        lax.fori_loop(0, lax.div(nb + 1, 2), pair, 0)

        @pl.when(nb >= 1)
        def _():
            wait_out(0)

        @pl.when(nb >= 2)
        def _():
            wait_out(1)
        plsc.subcore_barrier()

        def fin_rr(k):
            return jnp.minimum(r0 + k * B, NSEG - B)

        def fin_load(k, w):
            rr = fin_rr(k)
            pltpu.async_copy(acc.at[pl.ds(rr, B)], ymst[w], sem_in[w])
            pltpu.async_copy(aacc.at[pl.ds(rr, B)], astage[w], sem_in[w])

        def fin_load_wait(k, w):
            rr = fin_rr(k)
            pltpu.make_async_copy(acc.at[pl.ds(rr, B)], ymst[w],
                                  sem_in[w]).wait()
            pltpu.make_async_copy(aacc.at[pl.ds(rr, B)], astage[w],
                                  sem_in[w]).wait()

        def fin_store(k, w):
            rr = fin_rr(k)
            pltpu.async_copy(ymst[w], out_hbm.at[pl.ds(base + rr, B)],
                             sem_out[w])

        def fin_store_wait(k, w):
            rr = fin_rr(k)
            pltpu.make_async_copy(ymst[w], out_hbm.at[pl.ds(base + rr, B)],
                                  sem_out[w]).wait()

        fin_load(0, 0)
        for k in range(NCHUNK):
            w = k % 2
            fin_load_wait(k, w)
            if k + 1 < NCHUNK:
                if k >= 1:
                    fin_store_wait(k - 1, (k + 1) % 2)
                fin_load(k + 1, (k + 1) % 2)

            def fingrp(gi, _, w=w):
                av = astage[w][gi, pl.ds(0, L)]
                rv = jnp.float32(1.0) / av
                for j in range(D // L):
                    ymst[w][gi, pl.ds(j * L, L)] = (
                        ymst[w][gi, pl.ds(j * L, L)] * rv)
                return 0
            lax.fori_loop(0, B, fingrp, 0)
            fin_store(k, w)
        fin_store_wait(NCHUNK - 1, (NCHUNK - 1) % 2)
        if NCHUNK >= 2:
            fin_store_wait(NCHUNK - 2, (NCHUNK - 2) % 2)

    return sc_kernel


def kernel(X_in, e_map, v_count, Y, Y_att):
    E, D = Y.shape
    N = v_count.shape[0]
    att = Y_att.reshape((E,))
    cut = jnp.sum(e_map < jnp.int32(N // NC), dtype=jnp.int32)
    cut_arr = jnp.full((L,), cut, dtype=jnp.int32)
    return _build(E, N, D)(e_map, att, Y, cut_arr)
